# baseline (speedup 1.0000x reference)
import functools

import jax
import jax.numpy as jnp
from jax import lax
from jax.experimental import pallas as pl
from jax.experimental.pallas import tpu as pltpu
from jax.experimental.pallas import tpu_sc as plsc

N = 10000
E = 160000
NC = 2
NS = 16
NW = NC * NS
CHUNK = 128
CHUNKS_PER_TILE = 40
NCHUNK = NW * CHUNKS_PER_TILE
EPAD = NCHUNK * CHUNK
ROWS_PER_TILE = 640
NPAD = ROWS_PER_TILE * NS
NBUF = 2

_MESH = plsc.VectorSubcoreMesh(core_axis_name="c", subcore_axis_name="s",
                               num_cores=NC, num_subcores=NS)


def _zero_fill(buf):
    z16 = jnp.zeros((16,), jnp.float32)

    def zero_row(r, _):
        for j in range(8):
            buf[r, pl.ds(j * 16, 16)] = z16
        return 0

    lax.fori_loop(0, CHUNK, zero_row, 0)


def _writeback(acc, out, bufs, sems, tile_r0, out_r0):
    nk = ROWS_PER_TILE // CHUNK
    for k in range(nk):
        b = k % 2
        if k >= 2:
            pltpu.make_async_copy(acc.at[pl.ds(tile_r0, CHUNK)], bufs[b],
                                  sems[b]).wait()
        pltpu.sync_copy(acc.at[pl.ds(tile_r0 + k * CHUNK, CHUNK)], bufs[b])
        pltpu.async_copy(bufs[b], out.at[pl.ds(out_r0 + k * CHUNK, CHUNK)],
                         sems[b])
    for k in range(nk - 2, nk):
        b = k % 2
        pltpu.make_async_copy(acc.at[pl.ds(tile_r0, CHUNK)], bufs[b],
                              sems[b]).wait()


def _seg_sum_body(table, src, dst, out, acc, r0, r1, sidx, didx, g0, g1):
    c = lax.axis_index("c")
    s = lax.axis_index("s")
    w = s * NC + c
    rows = [r0, r1]
    gsem = [g0, g1]

    _zero_fill(r0)
    tile_r0 = pl.multiple_of(s * ROWS_PER_TILE, 128)
    for k in range(ROWS_PER_TILE // CHUNK):
        pltpu.sync_copy(r0, acc.at[pl.ds(tile_r0 + k * CHUNK, CHUNK)])

    row0 = pl.multiple_of(w * CHUNKS_PER_TILE, 8)
    pltpu.sync_copy(src.at[pl.ds(row0, CHUNKS_PER_TILE)], sidx)
    pltpu.sync_copy(dst.at[pl.ds(row0, CHUNKS_PER_TILE)], didx)
    plsc.subcore_barrier()

    def gather(j, b):
        pltpu.async_copy(table.at[sidx.at[j]], rows[b], gsem[b])

    def gather_wait(b):
        pltpu.make_async_copy(table.at[pl.ds(0, CHUNK)], rows[b],
                              gsem[b]).wait()

    def scatter(j, b):
        pltpu.sync_copy(rows[b], acc.at[didx.at[j]], add=True)

    gather(0, 0)

    def pipe_body(p, _):
        j0 = 2 * p
        gather_wait(0)
        gather(j0 + 1, 1)
        scatter(j0, 0)
        gather_wait(1)

        @pl.when(p < CHUNKS_PER_TILE // 2 - 1)
        def _():
            gather(j0 + 2, 0)

        scatter(j0 + 1, 1)
        return 0

    lax.fori_loop(0, CHUNKS_PER_TILE // 2, pipe_body, 0)
    plsc.subcore_barrier()

    _writeback(acc, out, [r0, r1], [g0, g1], tile_r0, c * NPAD + tile_r0)


@jax.jit
def _seg_sum(table, src, dst):
    return pl.kernel(
        _seg_sum_body,
        out_type=jax.ShapeDtypeStruct((NC * NPAD, 128), jnp.float32),
        mesh=_MESH,
        scratch_types=[
            pltpu.VMEM_SHARED((NPAD, 128), jnp.float32),
            pltpu.VMEM((CHUNK, 128), jnp.float32),
            pltpu.VMEM((CHUNK, 128), jnp.float32),
            pltpu.VMEM((CHUNKS_PER_TILE, CHUNK), jnp.int32),
            pltpu.VMEM((CHUNKS_PER_TILE, CHUNK), jnp.int32),
            pltpu.SemaphoreType.DMA,
            pltpu.SemaphoreType.DMA,
        ],
    )(table, src, dst)


def _deg_body(dst, out, acc, buf, stage0, didx, sem, o0, o1):
    c = lax.axis_index("c")
    s = lax.axis_index("s")
    w = s * NC + c

    _zero_fill(buf)
    tile_r0 = pl.multiple_of(s * ROWS_PER_TILE, 128)
    for k in range(ROWS_PER_TILE // CHUNK):
        pltpu.sync_copy(buf, acc.at[pl.ds(tile_r0 + k * CHUNK, CHUNK)])

    o16 = jnp.ones((16,), jnp.float32)

    def ones_row(r, _):
        for j in range(8):
            buf[r, pl.ds(j * 16, 16)] = o16
        return 0

    lax.fori_loop(0, CHUNK, ones_row, 0)
    row0 = pl.multiple_of(w * CHUNKS_PER_TILE, 8)
    pltpu.sync_copy(dst.at[pl.ds(row0, CHUNKS_PER_TILE)], didx)
    plsc.subcore_barrier()

    def pipe_body(p, _):
        for q in range(NBUF):
            pltpu.async_copy(buf, acc.at[didx.at[p * NBUF + q]], sem,
                             add=True)
        for q in range(NBUF):
            pltpu.make_async_copy(out.at[pl.ds(0, CHUNK)], buf, sem).wait()
        return 0

    lax.fori_loop(0, CHUNKS_PER_TILE // NBUF, pipe_body, 0)
    plsc.subcore_barrier()

    _writeback(acc, out, [buf, stage0], [o0, o1], tile_r0,
               c * NPAD + tile_r0)


@jax.jit
def _deg_count(dst):
    return pl.kernel(
        _deg_body,
        out_type=jax.ShapeDtypeStruct((NC * NPAD, 128), jnp.float32),
        mesh=_MESH,
        scratch_types=[
            pltpu.VMEM_SHARED((NPAD, 128), jnp.float32),
            pltpu.VMEM((CHUNK, 128), jnp.float32),
            pltpu.VMEM((CHUNK, 128), jnp.float32),
            pltpu.VMEM((CHUNKS_PER_TILE, CHUNK), jnp.int32),
            pltpu.SemaphoreType.DMA,
            pltpu.SemaphoreType.DMA,
            pltpu.SemaphoreType.DMA,
        ],
    )(dst)


_BN = 1000
_GRID = N // _BN


def _full(shape):
    return pl.BlockSpec(shape, lambda i: tuple(0 for _ in shape))


def _rows(shape):
    return pl.BlockSpec(shape, lambda i: (i,) + tuple(0 for _ in shape[1:]))


def _parts(shape):
    return pl.BlockSpec(shape, lambda i: (0, i, 0))


def _invdeg_body(dp_ref, out_ref):
    d = dp_ref[0] + dp_ref[1]
    out_ref[...] = (1.0 / jnp.clip(d, 1.0, None))[:, :16]


@jax.jit
def _invdeg(degp):
    return pl.pallas_call(
        _invdeg_body,
        grid=(_GRID,),
        in_specs=[_parts((NC, _BN, 128))],
        out_specs=_rows((_BN, 16)),
        out_shape=jax.ShapeDtypeStruct((N, 16), jnp.float32),
    )(degp)


def _dot(a, b):
    return jnp.dot(a, b, preferred_element_type=jnp.float32)


def _tc1_body(a0_ref, a1_ref, invd_ref, x_ref, w1l_ref, w1r_ref, b1_ref,
              w2l_ref, h1_ref, p2_ref):
    invd = invd_ref[:, 0:1]
    a0 = (a0_ref[0] + a0_ref[1]) * invd
    a1 = (a1_ref[0] + a1_ref[1]) * invd
    agg = jnp.concatenate([a0, a1], axis=1)
    h1 = jax.nn.relu(_dot(agg, w1l_ref[...]) + _dot(x_ref[...], w1r_ref[...])
                     + b1_ref[...])
    h1_ref[...] = h1
    p2_ref[...] = _dot(h1, w2l_ref[...])


@jax.jit
def _tc1(a0, a1, invd, x, W1l, W1r, b1, W2l):
    return pl.pallas_call(
        _tc1_body,
        grid=(_GRID,),
        in_specs=[_parts((NC, _BN, 128)), _parts((NC, _BN, 128)),
                  _rows((_BN, 16)), _rows((_BN, 256)),
                  _full((256, 256)), _full((256, 256)), _full((1, 256)),
                  _full((256, 128))],
        out_specs=[_rows((_BN, 256)), _rows((_BN, 128))],
        out_shape=[jax.ShapeDtypeStruct((N, 256), jnp.float32),
                   jax.ShapeDtypeStruct((N, 128), jnp.float32)],
    )(a0, a1, invd, x, W1l, W1r, b1, W2l)


def _tc2_body(ap_ref, invd_ref, h1_ref, w2r_ref, b2_ref, h2_ref):
    agg = (ap_ref[0] + ap_ref[1]) * invd_ref[:, 0:1]
    h2_ref[...] = jax.nn.relu(agg + _dot(h1_ref[...], w2r_ref[...])
                              + b2_ref[...])


@jax.jit
def _tc2(ap2, invd, h1, W2r, b2):
    return pl.pallas_call(
        _tc2_body,
        grid=(_GRID,),
        in_specs=[_parts((NC, _BN, 128)), _rows((_BN, 16)), _rows((_BN, 256)),
                  _full((256, 128)), _full((1, 128))],
        out_specs=_rows((_BN, 128)),
        out_shape=jax.ShapeDtypeStruct((N, 128), jnp.float32),
    )(ap2, invd, h1, W2r, b2)


def _tc3_body(ap_ref, invd_ref, h2_ref, w3l_ref, w3r_ref, b3_ref, w4l_ref,
              h3_ref, p4a_ref, p4b_ref):
    agg = (ap_ref[0] + ap_ref[1]) * invd_ref[:, 0:1]
    h3 = jax.nn.relu(_dot(agg, w3l_ref[...]) + _dot(h2_ref[...], w3r_ref[...])
                     + b3_ref[...])
    h3_ref[...] = h3
    p4 = _dot(h3, w4l_ref[...])
    p4a_ref[...] = p4[:, :128]
    p4b_ref[...] = p4[:, 128:]


@jax.jit
def _tc3(ah2, invd, h2, W3l, W3r, b3, W4l):
    return pl.pallas_call(
        _tc3_body,
        grid=(_GRID,),
        in_specs=[_parts((NC, _BN, 128)), _rows((_BN, 16)), _rows((_BN, 128)),
                  _full((128, 256)), _full((128, 256)), _full((1, 256)),
                  _full((256, 256))],
        out_specs=[_rows((_BN, 256)), _rows((_BN, 128)), _rows((_BN, 128))],
        out_shape=[jax.ShapeDtypeStruct((N, 256), jnp.float32),
                   jax.ShapeDtypeStruct((N, 128), jnp.float32),
                   jax.ShapeDtypeStruct((N, 128), jnp.float32)],
    )(ah2, invd, h2, W3l, W3r, b3, W4l)


def _tc4_body(a0_ref, a1_ref, invd_ref, h3_ref, w4r_ref, b4_ref, out_ref):
    invd = invd_ref[:, 0:1]
    a0 = (a0_ref[0] + a0_ref[1]) * invd
    a1 = (a1_ref[0] + a1_ref[1]) * invd
    agg = jnp.concatenate([a0, a1], axis=1)
    out_ref[...] = agg + _dot(h3_ref[...], w4r_ref[...]) + b4_ref[...]


@jax.jit
def _tc4(a4a, a4b, invd, h3, W4r, b4):
    return pl.pallas_call(
        _tc4_body,
        grid=(_GRID,),
        in_specs=[_parts((NC, _BN, 128)), _parts((NC, _BN, 128)),
                  _rows((_BN, 16)), _rows((_BN, 256)),
                  _full((256, 256)), _full((1, 256))],
        out_specs=_rows((_BN, 256)),
        out_shape=jax.ShapeDtypeStruct((N, 256), jnp.float32),
    )(a4a, a4b, invd, h3, W4r, b4)


def _partials(flat):
    return flat.reshape(NC, NPAD, -1)[:, :N, :]


def kernel(x, edge_index, W1l, W1r, b1, W2l, W2r, b2, W3l, W3r, b3,
           W4l, W4r, b4):
    src = edge_index[0].astype(jnp.int32)
    dst = edge_index[1].astype(jnp.int32)
    npad_e = EPAD - src.shape[0]
    src = jnp.concatenate([src, jnp.zeros((npad_e,), jnp.int32)])
    dst = jnp.concatenate([dst, jnp.full((npad_e,), NPAD - 1, jnp.int32)])
    src = src.reshape(NCHUNK, CHUNK)
    dst = dst.reshape(NCHUNK, CHUNK)

    degp = _partials(_deg_count(dst))
    invd = _invdeg(degp)

    xh0 = x[:, :128]
    xh1 = x[:, 128:]
    a0 = _partials(_seg_sum(xh0, src, dst))
    a1 = _partials(_seg_sum(xh1, src, dst))
    h1, p2 = _tc1(a0, a1, invd, x, W1l, W1r, b1.reshape(1, -1), W2l)

    ap2 = _partials(_seg_sum(p2, src, dst))
    h2 = _tc2(ap2, invd, h1, W2r, b2.reshape(1, -1))

    ah2 = _partials(_seg_sum(h2, src, dst))
    h3, p4a, p4b = _tc3(ah2, invd, h2, W3l, W3r, b3.reshape(1, -1), W4l)

    a4a = _partials(_seg_sum(p4a, src, dst))
    a4b = _partials(_seg_sum(p4b, src, dst))
    out = _tc4(a4a, a4b, invd, h3, W4r, b4.reshape(1, -1))
    return out


# speedup vs baseline: 2.8385x; 2.8385x over previous
import functools

import jax
import jax.numpy as jnp
from jax import lax
from jax.experimental import pallas as pl
from jax.experimental.pallas import tpu as pltpu
from jax.experimental.pallas import tpu_sc as plsc

N = 10000
E = 160000
NC = 2
NS = 16
NW = NC * NS
CHUNK = 128
CHUNKS_PER_TILE = 40
NCHUNK = NW * CHUNKS_PER_TILE
EPAD = NCHUNK * CHUNK
ROWS_PER_TILE = 640
NPAD = ROWS_PER_TILE * NS
NBUF = 2

_MESH = plsc.VectorSubcoreMesh(core_axis_name="c", subcore_axis_name="s",
                               num_cores=NC, num_subcores=NS)


def _zero_fill(buf):
    z16 = jnp.zeros((16,), jnp.float32)

    def zero_row(r, _):
        for j in range(8):
            buf[r, pl.ds(j * 16, 16)] = z16
        return 0

    lax.fori_loop(0, CHUNK, zero_row, 0)


def _writeback(acc, out, bufs, sems, tile_r0, out_r0):
    nk = ROWS_PER_TILE // CHUNK
    for k in range(nk):
        b = k % 2
        if k >= 2:
            pltpu.make_async_copy(acc.at[pl.ds(tile_r0, CHUNK)], bufs[b],
                                  sems[b]).wait()
        pltpu.sync_copy(acc.at[pl.ds(tile_r0 + k * CHUNK, CHUNK)], bufs[b])
        pltpu.async_copy(bufs[b], out.at[pl.ds(out_r0 + k * CHUNK, CHUNK)],
                         sems[b])
    for k in range(nk - 2, nk):
        b = k % 2
        pltpu.make_async_copy(acc.at[pl.ds(tile_r0, CHUNK)], bufs[b],
                              sems[b]).wait()


def _seg_sum_body(table, src, dst, out, acc, r0, r1, sidx, didx, g0, g1):
    c = lax.axis_index("c")
    s = lax.axis_index("s")
    w = s * NC + c
    rows = [r0, r1]
    gsem = [g0, g1]

    _zero_fill(r0)
    tile_r0 = pl.multiple_of(s * ROWS_PER_TILE, 128)
    for k in range(ROWS_PER_TILE // CHUNK):
        pltpu.sync_copy(r0, acc.at[pl.ds(tile_r0 + k * CHUNK, CHUNK)])

    row0 = pl.multiple_of(w * CHUNKS_PER_TILE, 8)
    pltpu.sync_copy(src.at[pl.ds(row0, CHUNKS_PER_TILE)], sidx)
    pltpu.sync_copy(dst.at[pl.ds(row0, CHUNKS_PER_TILE)], didx)
    plsc.subcore_barrier()

    def gather(j, b):
        pltpu.async_copy(table.at[sidx.at[j]], rows[b], gsem[b])

    def gather_wait(b):
        pltpu.make_async_copy(table.at[pl.ds(0, CHUNK)], rows[b],
                              gsem[b]).wait()

    def scatter(j, b):
        pltpu.sync_copy(rows[b], acc.at[didx.at[j]], add=True)

    gather(0, 0)

    def pipe_body(p, _):
        j0 = 2 * p
        gather_wait(0)
        gather(j0 + 1, 1)
        scatter(j0, 0)
        gather_wait(1)

        @pl.when(p < CHUNKS_PER_TILE // 2 - 1)
        def _():
            gather(j0 + 2, 0)

        scatter(j0 + 1, 1)
        return 0

    lax.fori_loop(0, CHUNKS_PER_TILE // 2, pipe_body, 0)
    plsc.subcore_barrier()

    _writeback(acc, out, [r0, r1], [g0, g1], tile_r0, c * NPAD + tile_r0)


@jax.jit
def _seg_sum(table, src, dst):
    return pl.kernel(
        _seg_sum_body,
        out_type=jax.ShapeDtypeStruct((NC * NPAD, 128), jnp.float32),
        mesh=_MESH,
        scratch_types=[
            pltpu.VMEM_SHARED((NPAD, 128), jnp.float32),
            pltpu.VMEM((CHUNK, 128), jnp.float32),
            pltpu.VMEM((CHUNK, 128), jnp.float32),
            pltpu.VMEM((CHUNKS_PER_TILE, CHUNK), jnp.int32),
            pltpu.VMEM((CHUNKS_PER_TILE, CHUNK), jnp.int32),
            pltpu.SemaphoreType.DMA,
            pltpu.SemaphoreType.DMA,
        ],
    )(table, src, dst)


def _deg_body(dst, out, acc, buf, stage0, didx, sem, o0, o1):
    c = lax.axis_index("c")
    s = lax.axis_index("s")
    w = s * NC + c

    _zero_fill(buf)
    tile_r0 = pl.multiple_of(s * ROWS_PER_TILE, 128)
    for k in range(ROWS_PER_TILE // CHUNK):
        pltpu.sync_copy(buf, acc.at[pl.ds(tile_r0 + k * CHUNK, CHUNK)])

    o16 = jnp.ones((16,), jnp.float32)

    def ones_row(r, _):
        for j in range(8):
            buf[r, pl.ds(j * 16, 16)] = o16
        return 0

    lax.fori_loop(0, CHUNK, ones_row, 0)
    row0 = pl.multiple_of(w * CHUNKS_PER_TILE, 8)
    pltpu.sync_copy(dst.at[pl.ds(row0, CHUNKS_PER_TILE)], didx)
    plsc.subcore_barrier()

    def pipe_body(p, _):
        for q in range(NBUF):
            pltpu.async_copy(buf, acc.at[didx.at[p * NBUF + q]], sem,
                             add=True)
        for q in range(NBUF):
            pltpu.make_async_copy(out.at[pl.ds(0, CHUNK)], buf, sem).wait()
        return 0

    lax.fori_loop(0, CHUNKS_PER_TILE // NBUF, pipe_body, 0)
    plsc.subcore_barrier()

    _writeback(acc, out, [buf, stage0], [o0, o1], tile_r0,
               c * NPAD + tile_r0)


@jax.jit
def _deg_count(dst):
    return pl.kernel(
        _deg_body,
        out_type=jax.ShapeDtypeStruct((NC * NPAD, 128), jnp.float32),
        mesh=_MESH,
        scratch_types=[
            pltpu.VMEM_SHARED((NPAD, 128), jnp.float32),
            pltpu.VMEM((CHUNK, 128), jnp.float32),
            pltpu.VMEM((CHUNK, 128), jnp.float32),
            pltpu.VMEM((CHUNKS_PER_TILE, CHUNK), jnp.int32),
            pltpu.SemaphoreType.DMA,
            pltpu.SemaphoreType.DMA,
            pltpu.SemaphoreType.DMA,
        ],
    )(dst)


_BN = 1000
_GRID = N // _BN


def _full(shape):
    return pl.BlockSpec(shape, lambda i: tuple(0 for _ in shape))


def _rows(shape):
    return pl.BlockSpec(shape, lambda i: (i,) + tuple(0 for _ in shape[1:]))


def _parts(shape):
    return pl.BlockSpec(shape, lambda i: (0, i, 0))


def _invdeg_body(dp_ref, out_ref):
    d = dp_ref[0] + dp_ref[1]
    out_ref[...] = (1.0 / jnp.clip(d, 1.0, None))[:, :16]


@jax.jit
def _invdeg(degp):
    return pl.pallas_call(
        _invdeg_body,
        grid=(_GRID,),
        in_specs=[_parts((NC, _BN, 128))],
        out_specs=_rows((_BN, 16)),
        out_shape=jax.ShapeDtypeStruct((N, 16), jnp.float32),
    )(degp)


def _dot(a, b):
    return jnp.dot(a, b, preferred_element_type=jnp.float32)


def _tc1_body(a0_ref, a1_ref, invd_ref, x_ref, w1l_ref, w1r_ref, b1_ref,
              w2l_ref, h1_ref, p2_ref):
    invd = invd_ref[:, 0:1]
    a0 = (a0_ref[0] + a0_ref[1]) * invd
    a1 = (a1_ref[0] + a1_ref[1]) * invd
    agg = jnp.concatenate([a0, a1], axis=1)
    h1 = jax.nn.relu(_dot(agg, w1l_ref[...]) + _dot(x_ref[...], w1r_ref[...])
                     + b1_ref[...])
    h1_ref[...] = h1
    p2_ref[...] = _dot(h1, w2l_ref[...])


@jax.jit
def _tc1(a0, a1, invd, x, W1l, W1r, b1, W2l):
    return pl.pallas_call(
        _tc1_body,
        grid=(_GRID,),
        in_specs=[_parts((NC, _BN, 128)), _parts((NC, _BN, 128)),
                  _rows((_BN, 16)), _rows((_BN, 256)),
                  _full((256, 256)), _full((256, 256)), _full((1, 256)),
                  _full((256, 128))],
        out_specs=[_rows((_BN, 256)), _rows((_BN, 128))],
        out_shape=[jax.ShapeDtypeStruct((N, 256), jnp.float32),
                   jax.ShapeDtypeStruct((N, 128), jnp.float32)],
    )(a0, a1, invd, x, W1l, W1r, b1, W2l)


def _tc2_body(ap_ref, invd_ref, h1_ref, w2r_ref, b2_ref, h2_ref):
    agg = (ap_ref[0] + ap_ref[1]) * invd_ref[:, 0:1]
    h2_ref[...] = jax.nn.relu(agg + _dot(h1_ref[...], w2r_ref[...])
                              + b2_ref[...])


@jax.jit
def _tc2(ap2, invd, h1, W2r, b2):
    return pl.pallas_call(
        _tc2_body,
        grid=(_GRID,),
        in_specs=[_parts((NC, _BN, 128)), _rows((_BN, 16)), _rows((_BN, 256)),
                  _full((256, 128)), _full((1, 128))],
        out_specs=_rows((_BN, 128)),
        out_shape=jax.ShapeDtypeStruct((N, 128), jnp.float32),
    )(ap2, invd, h1, W2r, b2)


def _tc3_body(ap_ref, invd_ref, h2_ref, w3l_ref, w3r_ref, b3_ref, w4l_ref,
              h3_ref, p4a_ref, p4b_ref):
    agg = (ap_ref[0] + ap_ref[1]) * invd_ref[:, 0:1]
    h3 = jax.nn.relu(_dot(agg, w3l_ref[...]) + _dot(h2_ref[...], w3r_ref[...])
                     + b3_ref[...])
    h3_ref[...] = h3
    p4 = _dot(h3, w4l_ref[...])
    p4a_ref[...] = p4[:, :128]
    p4b_ref[...] = p4[:, 128:]


@jax.jit
def _tc3(ah2, invd, h2, W3l, W3r, b3, W4l):
    return pl.pallas_call(
        _tc3_body,
        grid=(_GRID,),
        in_specs=[_parts((NC, _BN, 128)), _rows((_BN, 16)), _rows((_BN, 128)),
                  _full((128, 256)), _full((128, 256)), _full((1, 256)),
                  _full((256, 256))],
        out_specs=[_rows((_BN, 256)), _rows((_BN, 128)), _rows((_BN, 128))],
        out_shape=[jax.ShapeDtypeStruct((N, 256), jnp.float32),
                   jax.ShapeDtypeStruct((N, 128), jnp.float32),
                   jax.ShapeDtypeStruct((N, 128), jnp.float32)],
    )(ah2, invd, h2, W3l, W3r, b3, W4l)


def _tc4_body(a0_ref, a1_ref, invd_ref, h3_ref, w4r_ref, b4_ref, out_ref):
    invd = invd_ref[:, 0:1]
    a0 = (a0_ref[0] + a0_ref[1]) * invd
    a1 = (a1_ref[0] + a1_ref[1]) * invd
    agg = jnp.concatenate([a0, a1], axis=1)
    out_ref[...] = agg + _dot(h3_ref[...], w4r_ref[...]) + b4_ref[...]


@jax.jit
def _tc4(a4a, a4b, invd, h3, W4r, b4):
    return pl.pallas_call(
        _tc4_body,
        grid=(_GRID,),
        in_specs=[_parts((NC, _BN, 128)), _parts((NC, _BN, 128)),
                  _rows((_BN, 16)), _rows((_BN, 256)),
                  _full((256, 256)), _full((1, 256))],
        out_specs=_rows((_BN, 256)),
        out_shape=jax.ShapeDtypeStruct((N, 256), jnp.float32),
    )(a4a, a4b, invd, h3, W4r, b4)


def _partials(flat):
    return flat.reshape(NC, NPAD, -1)[:, :N, :]


def kernel(x, edge_index, W1l, W1r, b1, W2l, W2r, b2, W3l, W3r, b3,
           W4l, W4r, b4):
    src = edge_index[0].astype(jnp.int32)
    dst = edge_index[1].astype(jnp.int32)
    npad_e = EPAD - src.shape[0]
    pad_i = jnp.arange(npad_e, dtype=jnp.int32)
    src = jnp.concatenate([src, pad_i % N])
    dst = jnp.concatenate([dst, N + pad_i % (NPAD - N)])
    src = src.reshape(NCHUNK, CHUNK)
    dst = dst.reshape(NCHUNK, CHUNK)

    degp = _partials(_deg_count(dst))
    invd = _invdeg(degp)

    xh0 = x[:, :128]
    xh1 = x[:, 128:]
    a0 = _partials(_seg_sum(xh0, src, dst))
    a1 = _partials(_seg_sum(xh1, src, dst))
    h1, p2 = _tc1(a0, a1, invd, x, W1l, W1r, b1.reshape(1, -1), W2l)

    ap2 = _partials(_seg_sum(p2, src, dst))
    h2 = _tc2(ap2, invd, h1, W2r, b2.reshape(1, -1))

    ah2 = _partials(_seg_sum(h2, src, dst))
    h3, p4a, p4b = _tc3(ah2, invd, h2, W3l, W3r, b3.reshape(1, -1), W4l)

    a4a = _partials(_seg_sum(p4a, src, dst))
    a4b = _partials(_seg_sum(p4b, src, dst))
    out = _tc4(a4a, a4b, invd, h3, W4r, b4.reshape(1, -1))
    return out


# speedup vs baseline: 3.0298x; 1.0674x over previous
import functools

import jax
import jax.numpy as jnp
from jax import lax
from jax.experimental import pallas as pl
from jax.experimental.pallas import tpu as pltpu
from jax.experimental.pallas import tpu_sc as plsc

N = 10000
E = 160000
NC = 2
NS = 16
NW = NC * NS
CHUNK = 128
CHUNKS_PER_TILE = 40
NCHUNK = NW * CHUNKS_PER_TILE
EPAD = NCHUNK * CHUNK
ROWS_PER_TILE = 640
NPAD = ROWS_PER_TILE * NS
NBUF = 2

_MESH = plsc.VectorSubcoreMesh(core_axis_name="c", subcore_axis_name="s",
                               num_cores=NC, num_subcores=NS)


def _zero_fill(buf):
    z16 = jnp.zeros((16,), jnp.float32)

    def zero_row(r, _):
        for j in range(8):
            buf[r, pl.ds(j * 16, 16)] = z16
        return 0

    lax.fori_loop(0, CHUNK, zero_row, 0)


def _writeback(acc, out, bufs, sems, tile_r0, out_r0):
    nk = ROWS_PER_TILE // CHUNK
    for k in range(nk):
        b = k % 2
        if k >= 2:
            pltpu.make_async_copy(acc.at[pl.ds(tile_r0, CHUNK)], bufs[b],
                                  sems[b]).wait()
        pltpu.sync_copy(acc.at[pl.ds(tile_r0 + k * CHUNK, CHUNK)], bufs[b])
        pltpu.async_copy(bufs[b], out.at[pl.ds(out_r0 + k * CHUNK, CHUNK)],
                         sems[b])
    for k in range(nk - 2, nk):
        b = k % 2
        pltpu.make_async_copy(acc.at[pl.ds(tile_r0, CHUNK)], bufs[b],
                              sems[b]).wait()


def _seg_sum_body(table, src, dst, out, acc, r0, r1, sidx, didx, g0, g1):
    c = lax.axis_index("c")
    s = lax.axis_index("s")
    w = s * NC + c
    rows = [r0, r1]
    gsem = [g0, g1]

    _zero_fill(r0)
    tile_r0 = pl.multiple_of(s * ROWS_PER_TILE, 128)
    for k in range(ROWS_PER_TILE // CHUNK):
        pltpu.sync_copy(r0, acc.at[pl.ds(tile_r0 + k * CHUNK, CHUNK)])

    row0 = pl.multiple_of(w * CHUNKS_PER_TILE, 8)
    pltpu.sync_copy(src.at[pl.ds(row0, CHUNKS_PER_TILE)], sidx)
    pltpu.sync_copy(dst.at[pl.ds(row0, CHUNKS_PER_TILE)], didx)
    plsc.subcore_barrier()

    def gather(j, b):
        pltpu.async_copy(table.at[sidx.at[j]], rows[b], gsem[b])

    def gather_wait(b):
        pltpu.make_async_copy(table.at[pl.ds(0, CHUNK)], rows[b],
                              gsem[b]).wait()

    def scatter(j, b):
        pltpu.sync_copy(rows[b], acc.at[didx.at[j]], add=True)

    gather(0, 0)

    def pipe_body(p, _):
        j0 = 2 * p
        gather_wait(0)
        gather(j0 + 1, 1)
        scatter(j0, 0)
        gather_wait(1)

        @pl.when(p < CHUNKS_PER_TILE // 2 - 1)
        def _():
            gather(j0 + 2, 0)

        scatter(j0 + 1, 1)
        return 0

    lax.fori_loop(0, CHUNKS_PER_TILE // 2, pipe_body, 0)
    plsc.subcore_barrier()

    _writeback(acc, out, [r0, r1], [g0, g1], tile_r0, c * NPAD + tile_r0)


@jax.jit
def _seg_sum(table, src, dst):
    return pl.kernel(
        _seg_sum_body,
        out_type=jax.ShapeDtypeStruct((NC * NPAD, 128), jnp.float32),
        mesh=_MESH,
        scratch_types=[
            pltpu.VMEM_SHARED((NPAD, 128), jnp.float32),
            pltpu.VMEM((CHUNK, 128), jnp.float32),
            pltpu.VMEM((CHUNK, 128), jnp.float32),
            pltpu.VMEM((CHUNKS_PER_TILE, CHUNK), jnp.int32),
            pltpu.VMEM((CHUNKS_PER_TILE, CHUNK), jnp.int32),
            pltpu.SemaphoreType.DMA,
            pltpu.SemaphoreType.DMA,
        ],
    )(table, src, dst)


def _deg_body(dst, out, acc, buf, stage0, didx, sem, o0, o1):
    c = lax.axis_index("c")
    s = lax.axis_index("s")
    w = s * NC + c

    _zero_fill(buf)
    tile_r0 = pl.multiple_of(s * ROWS_PER_TILE, 128)
    for k in range(ROWS_PER_TILE // CHUNK):
        pltpu.sync_copy(buf, acc.at[pl.ds(tile_r0 + k * CHUNK, CHUNK)])

    o16 = jnp.ones((16,), jnp.float32)

    def ones_row(r, _):
        for j in range(8):
            buf[r, pl.ds(j * 16, 16)] = o16
        return 0

    lax.fori_loop(0, CHUNK, ones_row, 0)
    row0 = pl.multiple_of(w * CHUNKS_PER_TILE, 8)
    pltpu.sync_copy(dst.at[pl.ds(row0, CHUNKS_PER_TILE)], didx)
    plsc.subcore_barrier()

    def pipe_body(p, _):
        for q in range(NBUF):
            pltpu.async_copy(buf, acc.at[didx.at[p * NBUF + q]], sem,
                             add=True)
        for q in range(NBUF):
            pltpu.make_async_copy(out.at[pl.ds(0, CHUNK)], buf, sem).wait()
        return 0

    lax.fori_loop(0, CHUNKS_PER_TILE // NBUF, pipe_body, 0)
    plsc.subcore_barrier()

    _writeback(acc, out, [buf, stage0], [o0, o1], tile_r0,
               c * NPAD + tile_r0)


@jax.jit
def _deg_count(dst):
    return pl.kernel(
        _deg_body,
        out_type=jax.ShapeDtypeStruct((NC * NPAD, 128), jnp.float32),
        mesh=_MESH,
        scratch_types=[
            pltpu.VMEM_SHARED((NPAD, 128), jnp.float32),
            pltpu.VMEM((CHUNK, 128), jnp.float32),
            pltpu.VMEM((CHUNK, 128), jnp.float32),
            pltpu.VMEM((CHUNKS_PER_TILE, CHUNK), jnp.int32),
            pltpu.SemaphoreType.DMA,
            pltpu.SemaphoreType.DMA,
            pltpu.SemaphoreType.DMA,
        ],
    )(dst)


_BN = 1000
_GRID = N // _BN


def _full(shape):
    return pl.BlockSpec(shape, lambda i: tuple(0 for _ in shape))


def _rows(shape):
    return pl.BlockSpec(shape, lambda i: (i,) + tuple(0 for _ in shape[1:]))


def _parts(shape):
    return pl.BlockSpec(shape, lambda i: (0, i, 0))


def _dot(a, b):
    return jnp.dot(a, b, preferred_element_type=jnp.float32)


def _self_body(h_ref, w_ref, b_ref, y_ref):
    y_ref[...] = _dot(h_ref[...], w_ref[...]) + b_ref[...]


def _mk_self(din, dout):
    @jax.jit
    def f(h, W, b):
        return pl.pallas_call(
            _self_body,
            grid=(_GRID,),
            in_specs=[_rows((_BN, din)), _full((din, dout)), _full((1, dout))],
            out_specs=_rows((_BN, dout)),
            out_shape=jax.ShapeDtypeStruct((N, dout), jnp.float32),
        )(h, W, b)
    return f


_self_256_256 = _mk_self(256, 256)
_self_256_128 = _mk_self(256, 128)
_self_128_256 = _mk_self(128, 256)


def _tc1_body(dp_ref, a0_ref, a1_ref, y1_ref, w1l_ref, w2l_ref,
              h1_ref, p2_ref, invd_ref):
    d = dp_ref[0] + dp_ref[1]
    invd = (1.0 / jnp.clip(d, 1.0, None))[:, 0:1]
    invd_ref[...] = jnp.broadcast_to(invd, (invd.shape[0], 16))
    a0 = (a0_ref[0] + a0_ref[1]) * invd
    a1 = (a1_ref[0] + a1_ref[1]) * invd
    agg = jnp.concatenate([a0, a1], axis=1)
    h1 = jax.nn.relu(_dot(agg, w1l_ref[...]) + y1_ref[...])
    h1_ref[...] = h1
    p2_ref[...] = _dot(h1, w2l_ref[...])


@jax.jit
def _tc1(degp, a0, a1, y1, W1l, W2l):
    return pl.pallas_call(
        _tc1_body,
        grid=(_GRID,),
        in_specs=[_parts((NC, _BN, 128)), _parts((NC, _BN, 128)),
                  _parts((NC, _BN, 128)), _rows((_BN, 256)),
                  _full((256, 256)), _full((256, 128))],
        out_specs=[_rows((_BN, 256)), _rows((_BN, 128)), _rows((_BN, 16))],
        out_shape=[jax.ShapeDtypeStruct((N, 256), jnp.float32),
                   jax.ShapeDtypeStruct((N, 128), jnp.float32),
                   jax.ShapeDtypeStruct((N, 16), jnp.float32)],
    )(degp, a0, a1, y1, W1l, W2l)


def _tc2_body(ap_ref, invd_ref, y2_ref, h2_ref):
    agg = (ap_ref[0] + ap_ref[1]) * invd_ref[:, 0:1]
    h2_ref[...] = jax.nn.relu(agg + y2_ref[...])


@jax.jit
def _tc2(ap2, invd, y2):
    return pl.pallas_call(
        _tc2_body,
        grid=(_GRID,),
        in_specs=[_parts((NC, _BN, 128)), _rows((_BN, 16)),
                  _rows((_BN, 128))],
        out_specs=_rows((_BN, 128)),
        out_shape=jax.ShapeDtypeStruct((N, 128), jnp.float32),
    )(ap2, invd, y2)


def _tc3_body(ap_ref, invd_ref, y3_ref, w3l_ref, w4l_ref,
              h3_ref, p4a_ref, p4b_ref):
    agg = (ap_ref[0] + ap_ref[1]) * invd_ref[:, 0:1]
    h3 = jax.nn.relu(_dot(agg, w3l_ref[...]) + y3_ref[...])
    h3_ref[...] = h3
    p4 = _dot(h3, w4l_ref[...])
    p4a_ref[...] = p4[:, :128]
    p4b_ref[...] = p4[:, 128:]


@jax.jit
def _tc3(ah2, invd, y3, W3l, W4l):
    return pl.pallas_call(
        _tc3_body,
        grid=(_GRID,),
        in_specs=[_parts((NC, _BN, 128)), _rows((_BN, 16)),
                  _rows((_BN, 256)), _full((128, 256)), _full((256, 256))],
        out_specs=[_rows((_BN, 256)), _rows((_BN, 128)), _rows((_BN, 128))],
        out_shape=[jax.ShapeDtypeStruct((N, 256), jnp.float32),
                   jax.ShapeDtypeStruct((N, 128), jnp.float32),
                   jax.ShapeDtypeStruct((N, 128), jnp.float32)],
    )(ah2, invd, y3, W3l, W4l)


def _tc4_body(a0_ref, a1_ref, invd_ref, y4_ref, out_ref):
    invd = invd_ref[:, 0:1]
    a0 = (a0_ref[0] + a0_ref[1]) * invd
    a1 = (a1_ref[0] + a1_ref[1]) * invd
    agg = jnp.concatenate([a0, a1], axis=1)
    out_ref[...] = agg + y4_ref[...]


@jax.jit
def _tc4(a4a, a4b, invd, y4):
    return pl.pallas_call(
        _tc4_body,
        grid=(_GRID,),
        in_specs=[_parts((NC, _BN, 128)), _parts((NC, _BN, 128)),
                  _rows((_BN, 16)), _rows((_BN, 256))],
        out_specs=_rows((_BN, 256)),
        out_shape=jax.ShapeDtypeStruct((N, 256), jnp.float32),
    )(a4a, a4b, invd, y4)


def _parts3(flat):
    return flat.reshape(NC, NPAD, -1)


def kernel(x, edge_index, W1l, W1r, b1, W2l, W2r, b2, W3l, W3r, b3,
           W4l, W4r, b4):
    src = edge_index[0].astype(jnp.int32)
    dst = edge_index[1].astype(jnp.int32)
    npad_e = EPAD - src.shape[0]
    pad_i = jnp.arange(npad_e, dtype=jnp.int32)
    src = jnp.concatenate([src, pad_i % N])
    dst = jnp.concatenate([dst, N + pad_i % (NPAD - N)])
    src = src.reshape(NCHUNK, CHUNK)
    dst = dst.reshape(NCHUNK, CHUNK)

    xh0 = x[:, :128]
    xh1 = x[:, 128:]

    y1 = _self_256_256(x, W1r, b1.reshape(1, -1))
    degp = _parts3(_deg_count(dst))
    a0 = _parts3(_seg_sum(xh0, src, dst))
    a1 = _parts3(_seg_sum(xh1, src, dst))
    h1, p2, invd = _tc1(degp, a0, a1, y1, W1l, W2l)

    y2 = _self_256_128(h1, W2r, b2.reshape(1, -1))
    ap2 = _parts3(_seg_sum(p2, src, dst))
    h2 = _tc2(ap2, invd, y2)

    y3 = _self_128_256(h2, W3r, b3.reshape(1, -1))
    ah2 = _parts3(_seg_sum(h2, src, dst))
    h3, p4a, p4b = _tc3(ah2, invd, y3, W3l, W4l)

    y4 = _self_256_256(h3, W4r, b4.reshape(1, -1))
    a4a = _parts3(_seg_sum(p4a, src, dst))
    a4b = _parts3(_seg_sum(p4b, src, dst))
    out = _tc4(a4a, a4b, invd, y4)
    return out


# speedup vs baseline: 3.0859x; 1.0185x over previous
import functools

import jax
import jax.numpy as jnp
from jax import lax
from jax.experimental import pallas as pl
from jax.experimental.pallas import tpu as pltpu
from jax.experimental.pallas import tpu_sc as plsc

N = 10000
E = 160000
NC = 2
NS = 16
NW = NC * NS
CHUNK = 128
CHUNKS_PER_TILE = 40
NCHUNK = NW * CHUNKS_PER_TILE
EPAD = NCHUNK * CHUNK
ROWS_PER_TILE = 640
NPAD = ROWS_PER_TILE * NS
NBUF = 2

_MESH = plsc.VectorSubcoreMesh(core_axis_name="c", subcore_axis_name="s",
                               num_cores=NC, num_subcores=NS)


def _fill(buf, val, width):
    v16 = jnp.full((16,), val, jnp.float32)

    def fill_row(r, _):
        for j in range(width // 16):
            buf[r, pl.ds(j * 16, 16)] = v16
        return 0

    lax.fori_loop(0, CHUNK, fill_row, 0)


def _zero_acc_async(zbuf, acc, tile_r0, zsem, hbm_dummy):
    for k in range(ROWS_PER_TILE // CHUNK):
        pltpu.async_copy(zbuf, acc.at[pl.ds(tile_r0 + k * CHUNK, CHUNK)],
                         zsem)


def _zero_acc_drain(zbuf, zsem, hbm_dummy):
    for k in range(ROWS_PER_TILE // CHUNK):
        pltpu.make_async_copy(hbm_dummy, zbuf, zsem).wait()


def _writeback(acc, out, bufs, sems, tile_r0, out_r0):
    nk = ROWS_PER_TILE // CHUNK
    for k in range(nk):
        b = k % 2
        if k >= 2:
            pltpu.make_async_copy(acc.at[pl.ds(tile_r0, CHUNK)], bufs[b],
                                  sems[b]).wait()
        pltpu.sync_copy(acc.at[pl.ds(tile_r0 + k * CHUNK, CHUNK)], bufs[b])
        pltpu.async_copy(bufs[b], out.at[pl.ds(out_r0 + k * CHUNK, CHUNK)],
                         sems[b])
    for k in range(nk - 2, nk):
        b = k % 2
        pltpu.make_async_copy(acc.at[pl.ds(tile_r0, CHUNK)], bufs[b],
                              sems[b]).wait()


def _seg_sum_body(table, src, dst, out, acc, r0, r1, sidx, didx, g0, g1):
    c = lax.axis_index("c")
    s = lax.axis_index("s")
    w = s * NC + c
    rows = [r0, r1]
    gsem = [g0, g1]

    _fill(r0, 0.0, 128)
    tile_r0 = pl.multiple_of(s * ROWS_PER_TILE, 128)
    dummy = table.at[pl.ds(0, CHUNK)]
    _zero_acc_async(r0, acc, tile_r0, g0, dummy)

    row0 = pl.multiple_of(w * CHUNKS_PER_TILE, 8)
    pltpu.sync_copy(src.at[pl.ds(row0, CHUNKS_PER_TILE)], sidx)
    pltpu.sync_copy(dst.at[pl.ds(row0, CHUNKS_PER_TILE)], didx)
    _zero_acc_drain(r0, g0, dummy)
    plsc.subcore_barrier()

    def gather(j, b):
        pltpu.async_copy(table.at[sidx.at[j]], rows[b], gsem[b])

    def gather_wait(b):
        pltpu.make_async_copy(table.at[pl.ds(0, CHUNK)], rows[b],
                              gsem[b]).wait()

    def scatter(j, b):
        pltpu.sync_copy(rows[b], acc.at[didx.at[j]], add=True)

    gather(0, 0)

    def pipe_body(p, _):
        j0 = 2 * p
        gather_wait(0)
        gather(j0 + 1, 1)
        scatter(j0, 0)
        gather_wait(1)

        @pl.when(p < CHUNKS_PER_TILE // 2 - 1)
        def _():
            gather(j0 + 2, 0)

        scatter(j0 + 1, 1)
        return 0

    lax.fori_loop(0, CHUNKS_PER_TILE // 2, pipe_body, 0)
    plsc.subcore_barrier()

    _writeback(acc, out, [r0, r1], [g0, g1], tile_r0, c * NPAD + tile_r0)


@jax.jit
def _seg_sum(table, src, dst):
    return pl.kernel(
        _seg_sum_body,
        out_type=jax.ShapeDtypeStruct((NC * NPAD, 128), jnp.float32),
        mesh=_MESH,
        scratch_types=[
            pltpu.VMEM_SHARED((NPAD, 128), jnp.float32),
            pltpu.VMEM((CHUNK, 128), jnp.float32),
            pltpu.VMEM((CHUNK, 128), jnp.float32),
            pltpu.VMEM((CHUNKS_PER_TILE, CHUNK), jnp.int32),
            pltpu.VMEM((CHUNKS_PER_TILE, CHUNK), jnp.int32),
            pltpu.SemaphoreType.DMA,
            pltpu.SemaphoreType.DMA,
        ],
    )(table, src, dst)


DW = 128


def _deg_body(dst, out, acc, buf, stage0, didx, sem, o0, o1):
    c = lax.axis_index("c")
    s = lax.axis_index("s")
    w = s * NC + c

    _fill(buf, 0.0, DW)
    tile_r0 = pl.multiple_of(s * ROWS_PER_TILE, 128)
    dummy = out.at[pl.ds(0, CHUNK)]
    _zero_acc_async(buf, acc, tile_r0, sem, dummy)
    row0 = pl.multiple_of(w * CHUNKS_PER_TILE, 8)
    pltpu.sync_copy(dst.at[pl.ds(row0, CHUNKS_PER_TILE)], didx)
    _zero_acc_drain(buf, sem, dummy)
    _fill(buf, 1.0, DW)
    plsc.subcore_barrier()

    def pipe_body(p, _):
        for q in range(NBUF):
            pltpu.async_copy(buf, acc.at[didx.at[p * NBUF + q]], sem,
                             add=True)
        for q in range(NBUF):
            pltpu.make_async_copy(out.at[pl.ds(0, CHUNK)], buf, sem).wait()
        return 0

    lax.fori_loop(0, CHUNKS_PER_TILE // NBUF, pipe_body, 0)
    plsc.subcore_barrier()

    _writeback(acc, out, [buf, stage0], [o0, o1], tile_r0,
               c * NPAD + tile_r0)


@jax.jit
def _deg_count(dst):
    return pl.kernel(
        _deg_body,
        out_type=jax.ShapeDtypeStruct((NC * NPAD, DW), jnp.float32),
        mesh=_MESH,
        scratch_types=[
            pltpu.VMEM_SHARED((NPAD, DW), jnp.float32),
            pltpu.VMEM((CHUNK, DW), jnp.float32),
            pltpu.VMEM((CHUNK, DW), jnp.float32),
            pltpu.VMEM((CHUNKS_PER_TILE, CHUNK), jnp.int32),
            pltpu.SemaphoreType.DMA,
            pltpu.SemaphoreType.DMA,
            pltpu.SemaphoreType.DMA,
        ],
    )(dst)


_BN = 1000
_GRID = N // _BN


def _full(shape):
    return pl.BlockSpec(shape, lambda i: tuple(0 for _ in shape))


def _rows(shape):
    return pl.BlockSpec(shape, lambda i: (i,) + tuple(0 for _ in shape[1:]))


def _parts(shape):
    return pl.BlockSpec(shape, lambda i: (0, i, 0))


def _dot(a, b):
    return jnp.dot(a, b, preferred_element_type=jnp.float32)


def _self_body(h_ref, w_ref, b_ref, y_ref):
    y_ref[...] = _dot(h_ref[...], w_ref[...]) + b_ref[...]


def _mk_self(din, dout):
    @jax.jit
    def f(h, W, b):
        return pl.pallas_call(
            _self_body,
            grid=(_GRID,),
            in_specs=[_rows((_BN, din)), _full((din, dout)), _full((1, dout))],
            out_specs=_rows((_BN, dout)),
            out_shape=jax.ShapeDtypeStruct((N, dout), jnp.float32),
        )(h, W, b)
    return f


_self_256_256 = _mk_self(256, 256)
_self_256_128 = _mk_self(256, 128)
_self_128_256 = _mk_self(128, 256)


def _tc1_body(dp_ref, a0_ref, a1_ref, y1_ref, w1l_ref, w2l_ref,
              h1_ref, p2_ref, invd_ref):
    d = dp_ref[0] + dp_ref[1]
    invd = (1.0 / jnp.clip(d, 1.0, None))[:, 0:1]
    invd_ref[...] = jnp.broadcast_to(invd, (invd.shape[0], 16))
    a0 = (a0_ref[0] + a0_ref[1]) * invd
    a1 = (a1_ref[0] + a1_ref[1]) * invd
    agg = jnp.concatenate([a0, a1], axis=1)
    h1 = jax.nn.relu(_dot(agg, w1l_ref[...]) + y1_ref[...])
    h1_ref[...] = h1
    p2_ref[...] = _dot(h1, w2l_ref[...])


@jax.jit
def _tc1(degp, a0, a1, y1, W1l, W2l):
    return pl.pallas_call(
        _tc1_body,
        grid=(_GRID,),
        in_specs=[_parts((NC, _BN, DW)), _parts((NC, _BN, 128)),
                  _parts((NC, _BN, 128)), _rows((_BN, 256)),
                  _full((256, 256)), _full((256, 128))],
        out_specs=[_rows((_BN, 256)), _rows((_BN, 128)), _rows((_BN, 16))],
        out_shape=[jax.ShapeDtypeStruct((N, 256), jnp.float32),
                   jax.ShapeDtypeStruct((N, 128), jnp.float32),
                   jax.ShapeDtypeStruct((N, 16), jnp.float32)],
    )(degp, a0, a1, y1, W1l, W2l)


def _tc2_body(ap_ref, invd_ref, y2_ref, h2_ref):
    agg = (ap_ref[0] + ap_ref[1]) * invd_ref[:, 0:1]
    h2_ref[...] = jax.nn.relu(agg + y2_ref[...])


@jax.jit
def _tc2(ap2, invd, y2):
    return pl.pallas_call(
        _tc2_body,
        grid=(_GRID,),
        in_specs=[_parts((NC, _BN, 128)), _rows((_BN, 16)),
                  _rows((_BN, 128))],
        out_specs=_rows((_BN, 128)),
        out_shape=jax.ShapeDtypeStruct((N, 128), jnp.float32),
    )(ap2, invd, y2)


def _tc3_body(ap_ref, invd_ref, y3_ref, w3l_ref, w4l_ref,
              h3_ref, p4a_ref, p4b_ref):
    agg = (ap_ref[0] + ap_ref[1]) * invd_ref[:, 0:1]
    h3 = jax.nn.relu(_dot(agg, w3l_ref[...]) + y3_ref[...])
    h3_ref[...] = h3
    p4 = _dot(h3, w4l_ref[...])
    p4a_ref[...] = p4[:, :128]
    p4b_ref[...] = p4[:, 128:]


@jax.jit
def _tc3(ah2, invd, y3, W3l, W4l):
    return pl.pallas_call(
        _tc3_body,
        grid=(_GRID,),
        in_specs=[_parts((NC, _BN, 128)), _rows((_BN, 16)),
                  _rows((_BN, 256)), _full((128, 256)), _full((256, 256))],
        out_specs=[_rows((_BN, 256)), _rows((_BN, 128)), _rows((_BN, 128))],
        out_shape=[jax.ShapeDtypeStruct((N, 256), jnp.float32),
                   jax.ShapeDtypeStruct((N, 128), jnp.float32),
                   jax.ShapeDtypeStruct((N, 128), jnp.float32)],
    )(ah2, invd, y3, W3l, W4l)


def _tc4_body(a0_ref, a1_ref, invd_ref, y4_ref, out_ref):
    invd = invd_ref[:, 0:1]
    a0 = (a0_ref[0] + a0_ref[1]) * invd
    a1 = (a1_ref[0] + a1_ref[1]) * invd
    agg = jnp.concatenate([a0, a1], axis=1)
    out_ref[...] = agg + y4_ref[...]


@jax.jit
def _tc4(a4a, a4b, invd, y4):
    return pl.pallas_call(
        _tc4_body,
        grid=(_GRID,),
        in_specs=[_parts((NC, _BN, 128)), _parts((NC, _BN, 128)),
                  _rows((_BN, 16)), _rows((_BN, 256))],
        out_specs=_rows((_BN, 256)),
        out_shape=jax.ShapeDtypeStruct((N, 256), jnp.float32),
    )(a4a, a4b, invd, y4)


def _parts3(flat):
    return flat.reshape(NC, NPAD, -1)


def kernel(x, edge_index, W1l, W1r, b1, W2l, W2r, b2, W3l, W3r, b3,
           W4l, W4r, b4):
    src = edge_index[0].astype(jnp.int32)
    dst = edge_index[1].astype(jnp.int32)
    npad_e = EPAD - src.shape[0]
    pad_i = jnp.arange(npad_e, dtype=jnp.int32)
    src = jnp.concatenate([src, pad_i % N])
    dst = jnp.concatenate([dst, N + pad_i % (NPAD - N)])
    src = src.reshape(NCHUNK, CHUNK)
    dst = dst.reshape(NCHUNK, CHUNK)

    xh0 = x[:, :128]
    xh1 = x[:, 128:]

    y1 = _self_256_256(x, W1r, b1.reshape(1, -1))
    degp = _parts3(_deg_count(dst))
    a0 = _parts3(_seg_sum(xh0, src, dst))
    a1 = _parts3(_seg_sum(xh1, src, dst))
    h1, p2, invd = _tc1(degp, a0, a1, y1, W1l, W2l)

    y2 = _self_256_128(h1, W2r, b2.reshape(1, -1))
    ap2 = _parts3(_seg_sum(p2, src, dst))
    h2 = _tc2(ap2, invd, y2)

    y3 = _self_128_256(h2, W3r, b3.reshape(1, -1))
    ah2 = _parts3(_seg_sum(h2, src, dst))
    h3, p4a, p4b = _tc3(ah2, invd, y3, W3l, W4l)

    y4 = _self_256_256(h3, W4r, b4.reshape(1, -1))
    a4a = _parts3(_seg_sum(p4a, src, dst))
    a4b = _parts3(_seg_sum(p4b, src, dst))
    out = _tc4(a4a, a4b, invd, y4)
    return out


# speedup vs baseline: 3.2327x; 1.0476x over previous
import functools

import jax
import jax.numpy as jnp
from jax import lax
from jax.experimental import pallas as pl
from jax.experimental.pallas import tpu as pltpu
from jax.experimental.pallas import tpu_sc as plsc

N = 10000
E = 160000
NC = 2
NS = 16
NW = NC * NS
CHUNK = 128
CHUNKS_PER_TILE = 40
NCHUNK = NW * CHUNKS_PER_TILE
EPAD = NCHUNK * CHUNK
ROWS_PER_TILE = 640
NPAD = ROWS_PER_TILE * NS
NBUF = 2

_MESH = plsc.VectorSubcoreMesh(core_axis_name="c", subcore_axis_name="s",
                               num_cores=NC, num_subcores=NS)


def _fill(buf, val, width):
    v16 = jnp.full((16,), val, jnp.float32)

    def fill_row(r, _):
        for j in range(width // 16):
            buf[r, pl.ds(j * 16, 16)] = v16
        return 0

    lax.fori_loop(0, CHUNK, fill_row, 0)


def _zero_acc_async(zbuf, acc, tile_r0, zsem, hbm_dummy):
    for k in range(ROWS_PER_TILE // CHUNK):
        pltpu.async_copy(zbuf, acc.at[pl.ds(tile_r0 + k * CHUNK, CHUNK)],
                         zsem)


def _zero_acc_drain(zbuf, zsem, hbm_dummy):
    for k in range(ROWS_PER_TILE // CHUNK):
        pltpu.make_async_copy(hbm_dummy, zbuf, zsem).wait()


def _writeback(acc, out, bufs, sems, tile_r0, out_r0):
    nk = ROWS_PER_TILE // CHUNK
    for k in range(nk):
        b = k % 2
        if k >= 2:
            pltpu.make_async_copy(acc.at[pl.ds(tile_r0, CHUNK)], bufs[b],
                                  sems[b]).wait()
        pltpu.sync_copy(acc.at[pl.ds(tile_r0 + k * CHUNK, CHUNK)], bufs[b])
        pltpu.async_copy(bufs[b], out.at[pl.ds(out_r0 + k * CHUNK, CHUNK)],
                         sems[b])
    for k in range(nk - 2, nk):
        b = k % 2
        pltpu.make_async_copy(acc.at[pl.ds(tile_r0, CHUNK)], bufs[b],
                              sems[b]).wait()


def _seg_sum_body(table, src, dst, out, acc, r0, r1, sidx, didx, g0, g1):
    c = lax.axis_index("c")
    s = lax.axis_index("s")
    w = s * NC + c
    rows = [r0, r1]
    gsem = [g0, g1]

    _fill(r0, 0.0, 128)
    tile_r0 = pl.multiple_of(s * ROWS_PER_TILE, 128)
    dummy = table.at[pl.ds(0, CHUNK)]
    _zero_acc_async(r0, acc, tile_r0, g0, dummy)

    row0 = pl.multiple_of(w * CHUNKS_PER_TILE, 8)
    pltpu.sync_copy(src.at[pl.ds(row0, CHUNKS_PER_TILE)], sidx)
    pltpu.sync_copy(dst.at[pl.ds(row0, CHUNKS_PER_TILE)], didx)
    _zero_acc_drain(r0, g0, dummy)
    plsc.subcore_barrier()

    def gather(j, b):
        pltpu.async_copy(table.at[sidx.at[j]], rows[b], gsem[b])

    def gather_wait(b):
        pltpu.make_async_copy(table.at[pl.ds(0, CHUNK)], rows[b],
                              gsem[b]).wait()

    def scatter(j, b):
        pltpu.sync_copy(rows[b], acc.at[didx.at[j]], add=True)

    gather(0, 0)

    def pipe_body(p, _):
        j0 = 2 * p
        gather_wait(0)
        gather(j0 + 1, 1)
        scatter(j0, 0)
        gather_wait(1)

        @pl.when(p < CHUNKS_PER_TILE // 2 - 1)
        def _():
            gather(j0 + 2, 0)

        scatter(j0 + 1, 1)
        return 0

    lax.fori_loop(0, CHUNKS_PER_TILE // 2, pipe_body, 0)
    plsc.subcore_barrier()

    _writeback(acc, out, [r0, r1], [g0, g1], tile_r0, c * NPAD + tile_r0)


@jax.jit
def _seg_sum(table, src, dst):
    return pl.kernel(
        _seg_sum_body,
        out_type=jax.ShapeDtypeStruct((NC * NPAD, 128), jnp.float32),
        mesh=_MESH,
        scratch_types=[
            pltpu.VMEM_SHARED((NPAD, 128), jnp.float32),
            pltpu.VMEM((CHUNK, 128), jnp.float32),
            pltpu.VMEM((CHUNK, 128), jnp.float32),
            pltpu.VMEM((CHUNKS_PER_TILE, CHUNK), jnp.int32),
            pltpu.VMEM((CHUNKS_PER_TILE, CHUNK), jnp.int32),
            pltpu.SemaphoreType.DMA,
            pltpu.SemaphoreType.DMA,
        ],
    )(table, src, dst)


DW = 128


PIECE = 16


def _seg_sum2_body(t0, t1, src, dst, out, acc, r0, r1, sa0, da0, sa1, da1,
                   g0, g1, ps):
    c = lax.axis_index("c")
    s = lax.axis_index("s")
    rows = [r0, r1]
    gsem = [g0, g1]
    sidx = [sa0, sa1]
    didx = [da0, da1]

    _fill(r0, 0.0, 128)
    tile_r0 = pl.multiple_of(s * ROWS_PER_TILE, 128)
    dummy = t0.at[pl.ds(0, CHUNK)]
    _zero_acc_async(r0, acc, tile_r0, g0, dummy)
    base = pl.multiple_of(s * (CHUNKS_PER_TILE * NC), 8)
    pltpu.sync_copy(src.at[pl.ds(base, PIECE)], sa0)
    pltpu.sync_copy(dst.at[pl.ds(base, PIECE)], da0)
    _zero_acc_drain(r0, g0, dummy)
    plsc.subcore_barrier()

    def gather(ib, j, b):
        @pl.when(c == 0)
        def _():
            pltpu.async_copy(t0.at[sidx[ib].at[j]], rows[b], gsem[b])

        @pl.when(c == 1)
        def _():
            pltpu.async_copy(t1.at[sidx[ib].at[j]], rows[b], gsem[b])

    def gather_wait(b):
        pltpu.make_async_copy(dummy, rows[b], gsem[b]).wait()

    def scatter(ib, j, b):
        pltpu.sync_copy(rows[b], acc.at[didx[ib].at[j]], add=True)

    npieces = (CHUNKS_PER_TILE * NC) // PIECE
    for pc in range(npieces):
        ib = pc % 2
        nb = (pc + 1) % 2
        if pc + 1 < npieces:
            nxt = pl.multiple_of(base + (pc + 1) * PIECE, 4)
            pltpu.async_copy(src.at[pl.ds(nxt, PIECE)], sidx[nb], ps)
            pltpu.async_copy(dst.at[pl.ds(nxt, PIECE)], didx[nb], ps)
        gather(ib, 0, 0)

        def pipe_body(q, _):
            j0 = 2 * q
            gather_wait(0)
            gather(ib, j0 + 1, 1)
            scatter(ib, j0, 0)
            gather_wait(1)

            @pl.when(q < PIECE // 2 - 1)
            def _():
                gather(ib, j0 + 2, 0)

            scatter(ib, j0 + 1, 1)
            return 0

        lax.fori_loop(0, PIECE // 2, pipe_body, 0)
        if pc + 1 < npieces:
            pltpu.make_async_copy(src.at[pl.ds(0, PIECE)], sidx[nb],
                                  ps).wait()
            pltpu.make_async_copy(src.at[pl.ds(0, PIECE)], didx[nb],
                                  ps).wait()
    plsc.subcore_barrier()

    _writeback(acc, out, [r0, r1], [g0, g1], tile_r0, c * NPAD + tile_r0)


@jax.jit
def _seg_sum2(t0, t1, src, dst):
    return pl.kernel(
        _seg_sum2_body,
        out_type=jax.ShapeDtypeStruct((NC * NPAD, 128), jnp.float32),
        mesh=_MESH,
        scratch_types=[
            pltpu.VMEM_SHARED((NPAD, 128), jnp.float32),
            pltpu.VMEM((CHUNK, 128), jnp.float32),
            pltpu.VMEM((CHUNK, 128), jnp.float32),
            pltpu.VMEM((PIECE, CHUNK), jnp.int32),
            pltpu.VMEM((PIECE, CHUNK), jnp.int32),
            pltpu.VMEM((PIECE, CHUNK), jnp.int32),
            pltpu.VMEM((PIECE, CHUNK), jnp.int32),
            pltpu.SemaphoreType.DMA,
            pltpu.SemaphoreType.DMA,
            pltpu.SemaphoreType.DMA,
        ],
    )(t0, t1, src, dst)


def _deg_body(dst, out, acc, buf, stage0, didx, sem, o0, o1):
    c = lax.axis_index("c")
    s = lax.axis_index("s")
    w = s * NC + c

    _fill(buf, 0.0, DW)
    tile_r0 = pl.multiple_of(s * ROWS_PER_TILE, 128)
    dummy = out.at[pl.ds(0, CHUNK)]
    _zero_acc_async(buf, acc, tile_r0, sem, dummy)
    row0 = pl.multiple_of(w * CHUNKS_PER_TILE, 8)
    pltpu.sync_copy(dst.at[pl.ds(row0, CHUNKS_PER_TILE)], didx)
    _zero_acc_drain(buf, sem, dummy)
    _fill(buf, 1.0, DW)
    plsc.subcore_barrier()

    def pipe_body(p, _):
        for q in range(NBUF):
            pltpu.async_copy(buf, acc.at[didx.at[p * NBUF + q]], sem,
                             add=True)
        for q in range(NBUF):
            pltpu.make_async_copy(out.at[pl.ds(0, CHUNK)], buf, sem).wait()
        return 0

    lax.fori_loop(0, CHUNKS_PER_TILE // NBUF, pipe_body, 0)
    plsc.subcore_barrier()

    _writeback(acc, out, [buf, stage0], [o0, o1], tile_r0,
               c * NPAD + tile_r0)


@jax.jit
def _deg_count(dst):
    return pl.kernel(
        _deg_body,
        out_type=jax.ShapeDtypeStruct((NC * NPAD, DW), jnp.float32),
        mesh=_MESH,
        scratch_types=[
            pltpu.VMEM_SHARED((NPAD, DW), jnp.float32),
            pltpu.VMEM((CHUNK, DW), jnp.float32),
            pltpu.VMEM((CHUNK, DW), jnp.float32),
            pltpu.VMEM((CHUNKS_PER_TILE, CHUNK), jnp.int32),
            pltpu.SemaphoreType.DMA,
            pltpu.SemaphoreType.DMA,
            pltpu.SemaphoreType.DMA,
        ],
    )(dst)


_BN = 1000
_GRID = N // _BN


def _full(shape):
    return pl.BlockSpec(shape, lambda i: tuple(0 for _ in shape))


def _rows(shape):
    return pl.BlockSpec(shape, lambda i: (i,) + tuple(0 for _ in shape[1:]))


def _parts(shape):
    return pl.BlockSpec(shape, lambda i: (0, i, 0))


def _dot(a, b):
    return jnp.dot(a, b, preferred_element_type=jnp.float32)


def _self_body(h_ref, w_ref, b_ref, y_ref):
    y_ref[...] = _dot(h_ref[...], w_ref[...]) + b_ref[...]


def _mk_self(din, dout):
    @jax.jit
    def f(h, W, b):
        return pl.pallas_call(
            _self_body,
            grid=(_GRID,),
            in_specs=[_rows((_BN, din)), _full((din, dout)), _full((1, dout))],
            out_specs=_rows((_BN, dout)),
            out_shape=jax.ShapeDtypeStruct((N, dout), jnp.float32),
        )(h, W, b)
    return f


_self_256_256 = _mk_self(256, 256)
_self_256_128 = _mk_self(256, 128)
_self_128_256 = _mk_self(128, 256)


def _tc1_body(dp_ref, ax_ref, y1_ref, w1l_ref, w2l_ref,
              h1_ref, p2_ref, invd_ref):
    d = dp_ref[0] + dp_ref[1]
    invd = (1.0 / jnp.clip(d, 1.0, None))[:, 0:1]
    invd_ref[...] = jnp.broadcast_to(invd, (invd.shape[0], 16))
    agg = jnp.concatenate([ax_ref[0] * invd, ax_ref[1] * invd], axis=1)
    h1 = jax.nn.relu(_dot(agg, w1l_ref[...]) + y1_ref[...])
    h1_ref[...] = h1
    p2_ref[...] = _dot(h1, w2l_ref[...])


@jax.jit
def _tc1(degp, aggx, y1, W1l, W2l):
    return pl.pallas_call(
        _tc1_body,
        grid=(_GRID,),
        in_specs=[_parts((NC, _BN, DW)), _parts((NC, _BN, 128)),
                  _rows((_BN, 256)), _full((256, 256)), _full((256, 128))],
        out_specs=[_rows((_BN, 256)), _rows((_BN, 128)), _rows((_BN, 16))],
        out_shape=[jax.ShapeDtypeStruct((N, 256), jnp.float32),
                   jax.ShapeDtypeStruct((N, 128), jnp.float32),
                   jax.ShapeDtypeStruct((N, 16), jnp.float32)],
    )(degp, aggx, y1, W1l, W2l)


def _tc2_body(ap_ref, invd_ref, y2_ref, h2_ref):
    agg = (ap_ref[0] + ap_ref[1]) * invd_ref[:, 0:1]
    h2_ref[...] = jax.nn.relu(agg + y2_ref[...])


@jax.jit
def _tc2(ap2, invd, y2):
    return pl.pallas_call(
        _tc2_body,
        grid=(_GRID,),
        in_specs=[_parts((NC, _BN, 128)), _rows((_BN, 16)),
                  _rows((_BN, 128))],
        out_specs=_rows((_BN, 128)),
        out_shape=jax.ShapeDtypeStruct((N, 128), jnp.float32),
    )(ap2, invd, y2)


def _tc3_body(ap_ref, invd_ref, y3_ref, w3l_ref, w4l_ref,
              h3_ref, p4a_ref, p4b_ref):
    agg = (ap_ref[0] + ap_ref[1]) * invd_ref[:, 0:1]
    h3 = jax.nn.relu(_dot(agg, w3l_ref[...]) + y3_ref[...])
    h3_ref[...] = h3
    p4 = _dot(h3, w4l_ref[...])
    p4a_ref[...] = p4[:, :128]
    p4b_ref[...] = p4[:, 128:]


@jax.jit
def _tc3(ah2, invd, y3, W3l, W4l):
    return pl.pallas_call(
        _tc3_body,
        grid=(_GRID,),
        in_specs=[_parts((NC, _BN, 128)), _rows((_BN, 16)),
                  _rows((_BN, 256)), _full((128, 256)), _full((256, 256))],
        out_specs=[_rows((_BN, 256)), _rows((_BN, 128)), _rows((_BN, 128))],
        out_shape=[jax.ShapeDtypeStruct((N, 256), jnp.float32),
                   jax.ShapeDtypeStruct((N, 128), jnp.float32),
                   jax.ShapeDtypeStruct((N, 128), jnp.float32)],
    )(ah2, invd, y3, W3l, W4l)


def _tc4_body(ax_ref, invd_ref, y4_ref, out_ref):
    invd = invd_ref[:, 0:1]
    agg = jnp.concatenate([ax_ref[0] * invd, ax_ref[1] * invd], axis=1)
    out_ref[...] = agg + y4_ref[...]


@jax.jit
def _tc4(agg4, invd, y4):
    return pl.pallas_call(
        _tc4_body,
        grid=(_GRID,),
        in_specs=[_parts((NC, _BN, 128)), _rows((_BN, 16)),
                  _rows((_BN, 256))],
        out_specs=_rows((_BN, 256)),
        out_shape=jax.ShapeDtypeStruct((N, 256), jnp.float32),
    )(agg4, invd, y4)


def _parts3(flat):
    return flat.reshape(NC, NPAD, -1)


def kernel(x, edge_index, W1l, W1r, b1, W2l, W2r, b2, W3l, W3r, b3,
           W4l, W4r, b4):
    src = edge_index[0].astype(jnp.int32)
    dst = edge_index[1].astype(jnp.int32)
    npad_e = EPAD - src.shape[0]
    pad_i = jnp.arange(npad_e, dtype=jnp.int32)
    src = jnp.concatenate([src, pad_i % N])
    dst = jnp.concatenate([dst, N + pad_i % (NPAD - N)])
    src = src.reshape(NCHUNK, CHUNK)
    dst = dst.reshape(NCHUNK, CHUNK)

    xh0 = x[:, :128]
    xh1 = x[:, 128:]

    y1 = _self_256_256(x, W1r, b1.reshape(1, -1))
    degp = _parts3(_deg_count(dst))
    aggx = _parts3(_seg_sum2(xh0, xh1, src, dst))
    h1, p2, invd = _tc1(degp, aggx, y1, W1l, W2l)

    y2 = _self_256_128(h1, W2r, b2.reshape(1, -1))
    ap2 = _parts3(_seg_sum(p2, src, dst))
    h2 = _tc2(ap2, invd, y2)

    y3 = _self_128_256(h2, W3r, b3.reshape(1, -1))
    ah2 = _parts3(_seg_sum(h2, src, dst))
    h3, p4a, p4b = _tc3(ah2, invd, y3, W3l, W4l)

    y4 = _self_256_256(h3, W4r, b4.reshape(1, -1))
    agg4 = _parts3(_seg_sum2(p4a, p4b, src, dst))
    out = _tc4(agg4, invd, y4)
    return out


# speedup vs baseline: 3.2493x; 1.0052x over previous
import functools

import numpy as np
import jax
import jax.numpy as jnp
from jax import lax
from jax.experimental import pallas as pl
from jax.experimental.pallas import tpu as pltpu
from jax.experimental.pallas import tpu_sc as plsc

N = 10000
E = 160000
NC = 2
NS = 16
NW = NC * NS
CHUNK = 128
CHUNKS_PER_TILE = 40
NCHUNK = NW * CHUNKS_PER_TILE
EPAD = NCHUNK * CHUNK
ROWS_PER_TILE = 640
NPAD = ROWS_PER_TILE * NS
NBUF = 2

_MESH = plsc.VectorSubcoreMesh(core_axis_name="c", subcore_axis_name="s",
                               num_cores=NC, num_subcores=NS)


def _fill(buf, val, width):
    v16 = jnp.full((16,), val, jnp.float32)

    def fill_row(r, _):
        for j in range(width // 16):
            buf[r, pl.ds(j * 16, 16)] = v16
        return 0

    lax.fori_loop(0, CHUNK, fill_row, 0)


def _zero_acc_async(zbuf, acc, tile_r0, zsem, hbm_dummy):
    for k in range(ROWS_PER_TILE // CHUNK):
        pltpu.async_copy(zbuf, acc.at[pl.ds(tile_r0 + k * CHUNK, CHUNK)],
                         zsem)


def _zero_acc_drain(zbuf, zsem, hbm_dummy):
    for k in range(ROWS_PER_TILE // CHUNK):
        pltpu.make_async_copy(hbm_dummy, zbuf, zsem).wait()


def _writeback(acc, out, bufs, sems, tile_r0, out_r0):
    nk = ROWS_PER_TILE // CHUNK
    for k in range(nk):
        b = k % 2
        if k >= 2:
            pltpu.make_async_copy(acc.at[pl.ds(tile_r0, CHUNK)], bufs[b],
                                  sems[b]).wait()
        pltpu.sync_copy(acc.at[pl.ds(tile_r0 + k * CHUNK, CHUNK)], bufs[b])
        pltpu.async_copy(bufs[b], out.at[pl.ds(out_r0 + k * CHUNK, CHUNK)],
                         sems[b])
    for k in range(nk - 2, nk):
        b = k % 2
        pltpu.make_async_copy(acc.at[pl.ds(tile_r0, CHUNK)], bufs[b],
                              sems[b]).wait()


def _seg_sum_body(table, src, dst, out, acc, r0, r1, sidx, didx, g0, g1):
    c = lax.axis_index("c")
    s = lax.axis_index("s")
    w = s * NC + c
    rows = [r0, r1]
    gsem = [g0, g1]

    _fill(r0, 0.0, 128)
    tile_r0 = pl.multiple_of(s * ROWS_PER_TILE, 128)
    dummy = table.at[pl.ds(0, CHUNK)]
    _zero_acc_async(r0, acc, tile_r0, g0, dummy)

    row0 = pl.multiple_of(w * CHUNKS_PER_TILE, 8)
    pltpu.sync_copy(src.at[pl.ds(row0, CHUNKS_PER_TILE)], sidx)
    pltpu.sync_copy(dst.at[pl.ds(row0, CHUNKS_PER_TILE)], didx)
    _zero_acc_drain(r0, g0, dummy)
    plsc.subcore_barrier()

    def gather(j, b):
        pltpu.async_copy(table.at[sidx.at[j]], rows[b], gsem[b])

    def gather_wait(b):
        pltpu.make_async_copy(table.at[pl.ds(0, CHUNK)], rows[b],
                              gsem[b]).wait()

    def scatter(j, b):
        pltpu.sync_copy(rows[b], acc.at[didx.at[j]], add=True)

    gather(0, 0)

    def pipe_body(p, _):
        j0 = 2 * p
        gather_wait(0)
        gather(j0 + 1, 1)
        scatter(j0, 0)
        gather_wait(1)

        @pl.when(p < CHUNKS_PER_TILE // 2 - 1)
        def _():
            gather(j0 + 2, 0)

        scatter(j0 + 1, 1)
        return 0

    lax.fori_loop(0, CHUNKS_PER_TILE // 2, pipe_body, 0)
    plsc.subcore_barrier()

    _writeback(acc, out, [r0, r1], [g0, g1], tile_r0, c * NPAD + tile_r0)


@jax.jit
def _seg_sum(table, src, dst):
    return pl.kernel(
        _seg_sum_body,
        out_type=jax.ShapeDtypeStruct((NC * NPAD, 128), jnp.float32),
        mesh=_MESH,
        scratch_types=[
            pltpu.VMEM_SHARED((NPAD, 128), jnp.float32),
            pltpu.VMEM((CHUNK, 128), jnp.float32),
            pltpu.VMEM((CHUNK, 128), jnp.float32),
            pltpu.VMEM((CHUNKS_PER_TILE, CHUNK), jnp.int32),
            pltpu.VMEM((CHUNKS_PER_TILE, CHUNK), jnp.int32),
            pltpu.SemaphoreType.DMA,
            pltpu.SemaphoreType.DMA,
        ],
    )(table, src, dst)


DW = 128


PIECE = 16


def _seg_sum2_body(t0, t1, src, dst, out, acc, r0, r1, sa0, da0, sa1, da1,
                   g0, g1, ps):
    c = lax.axis_index("c")
    s = lax.axis_index("s")
    rows = [r0, r1]
    gsem = [g0, g1]
    sidx = [sa0, sa1]
    didx = [da0, da1]

    _fill(r0, 0.0, 128)
    tile_r0 = pl.multiple_of(s * ROWS_PER_TILE, 128)
    dummy = t0.at[pl.ds(0, CHUNK)]
    _zero_acc_async(r0, acc, tile_r0, g0, dummy)
    base = pl.multiple_of(s * (CHUNKS_PER_TILE * NC), 8)
    pltpu.sync_copy(src.at[pl.ds(base, PIECE)], sa0)
    pltpu.sync_copy(dst.at[pl.ds(base, PIECE)], da0)
    _zero_acc_drain(r0, g0, dummy)
    plsc.subcore_barrier()

    def gather(ib, j, b):
        @pl.when(c == 0)
        def _():
            pltpu.async_copy(t0.at[sidx[ib].at[j]], rows[b], gsem[b])

        @pl.when(c == 1)
        def _():
            pltpu.async_copy(t1.at[sidx[ib].at[j]], rows[b], gsem[b])

    def gather_wait(b):
        pltpu.make_async_copy(dummy, rows[b], gsem[b]).wait()

    def scatter(ib, j, b):
        pltpu.sync_copy(rows[b], acc.at[didx[ib].at[j]], add=True)

    npieces = (CHUNKS_PER_TILE * NC) // PIECE
    for pc in range(npieces):
        ib = pc % 2
        nb = (pc + 1) % 2
        if pc + 1 < npieces:
            nxt = pl.multiple_of(base + (pc + 1) * PIECE, 4)
            pltpu.async_copy(src.at[pl.ds(nxt, PIECE)], sidx[nb], ps)
            pltpu.async_copy(dst.at[pl.ds(nxt, PIECE)], didx[nb], ps)
        gather(ib, 0, 0)

        def pipe_body(q, _):
            j0 = 2 * q
            gather_wait(0)
            gather(ib, j0 + 1, 1)
            scatter(ib, j0, 0)
            gather_wait(1)

            @pl.when(q < PIECE // 2 - 1)
            def _():
                gather(ib, j0 + 2, 0)

            scatter(ib, j0 + 1, 1)
            return 0

        lax.fori_loop(0, PIECE // 2, pipe_body, 0)
        if pc + 1 < npieces:
            pltpu.make_async_copy(src.at[pl.ds(0, PIECE)], sidx[nb],
                                  ps).wait()
            pltpu.make_async_copy(src.at[pl.ds(0, PIECE)], didx[nb],
                                  ps).wait()
    plsc.subcore_barrier()

    _writeback(acc, out, [r0, r1], [g0, g1], tile_r0, c * NPAD + tile_r0)


@jax.jit
def _seg_sum2(t0, t1, src, dst):
    return pl.kernel(
        _seg_sum2_body,
        out_type=jax.ShapeDtypeStruct((NC * NPAD, 128), jnp.float32),
        mesh=_MESH,
        scratch_types=[
            pltpu.VMEM_SHARED((NPAD, 128), jnp.float32),
            pltpu.VMEM((CHUNK, 128), jnp.float32),
            pltpu.VMEM((CHUNK, 128), jnp.float32),
            pltpu.VMEM((PIECE, CHUNK), jnp.int32),
            pltpu.VMEM((PIECE, CHUNK), jnp.int32),
            pltpu.VMEM((PIECE, CHUNK), jnp.int32),
            pltpu.VMEM((PIECE, CHUNK), jnp.int32),
            pltpu.SemaphoreType.DMA,
            pltpu.SemaphoreType.DMA,
            pltpu.SemaphoreType.DMA,
        ],
    )(t0, t1, src, dst)


def _deg_body(dst, out, acc, buf, stage0, didx, sem, o0, o1):
    c = lax.axis_index("c")
    s = lax.axis_index("s")
    w = s * NC + c

    _fill(buf, 0.0, DW)
    tile_r0 = pl.multiple_of(s * ROWS_PER_TILE, 128)
    dummy = out.at[pl.ds(0, CHUNK)]
    _zero_acc_async(buf, acc, tile_r0, sem, dummy)
    row0 = pl.multiple_of(w * CHUNKS_PER_TILE, 8)
    pltpu.sync_copy(dst.at[pl.ds(row0, CHUNKS_PER_TILE)], didx)
    _zero_acc_drain(buf, sem, dummy)
    _fill(buf, 1.0, DW)
    plsc.subcore_barrier()

    def pipe_body(p, _):
        for q in range(NBUF):
            pltpu.async_copy(buf, acc.at[didx.at[p * NBUF + q]], sem,
                             add=True)
        for q in range(NBUF):
            pltpu.make_async_copy(out.at[pl.ds(0, CHUNK)], buf, sem).wait()
        return 0

    lax.fori_loop(0, CHUNKS_PER_TILE // NBUF, pipe_body, 0)
    plsc.subcore_barrier()

    _writeback(acc, out, [buf, stage0], [o0, o1], tile_r0,
               c * NPAD + tile_r0)


@jax.jit
def _deg_count(dst):
    return pl.kernel(
        _deg_body,
        out_type=jax.ShapeDtypeStruct((NC * NPAD, DW), jnp.float32),
        mesh=_MESH,
        scratch_types=[
            pltpu.VMEM_SHARED((NPAD, DW), jnp.float32),
            pltpu.VMEM((CHUNK, DW), jnp.float32),
            pltpu.VMEM((CHUNK, DW), jnp.float32),
            pltpu.VMEM((CHUNKS_PER_TILE, CHUNK), jnp.int32),
            pltpu.SemaphoreType.DMA,
            pltpu.SemaphoreType.DMA,
            pltpu.SemaphoreType.DMA,
        ],
    )(dst)


_BN = 1000
_GRID = N // _BN


def _full(shape):
    return pl.BlockSpec(shape, lambda i: tuple(0 for _ in shape))


def _rows(shape):
    return pl.BlockSpec(shape, lambda i: (i,) + tuple(0 for _ in shape[1:]))


def _parts(shape):
    return pl.BlockSpec(shape, lambda i: (0, i, 0))


def _dot(a, b):
    return jnp.dot(a, b, preferred_element_type=jnp.float32)


def _self_body(h_ref, w_ref, b_ref, y_ref):
    y_ref[...] = _dot(h_ref[...], w_ref[...]) + b_ref[...]


def _mk_self(din, dout):
    @jax.jit
    def f(h, W, b):
        return pl.pallas_call(
            _self_body,
            grid=(_GRID,),
            in_specs=[_rows((_BN, din)), _full((din, dout)), _full((1, dout))],
            out_specs=_rows((_BN, dout)),
            out_shape=jax.ShapeDtypeStruct((N, dout), jnp.float32),
        )(h, W, b)
    return f


_self_256_256 = _mk_self(256, 256)
_self_256_128 = _mk_self(256, 128)
_self_128_256 = _mk_self(128, 256)


def _tc1_body(dp_ref, ax_ref, y1_ref, w1l_ref, w2l_ref,
              h1_ref, p2_ref, invd_ref):
    d = dp_ref[0] + dp_ref[1]
    invd = (1.0 / jnp.clip(d, 1.0, None))[:, 0:1]
    invd_ref[...] = jnp.broadcast_to(invd, (invd.shape[0], 16))
    agg = jnp.concatenate([ax_ref[0] * invd, ax_ref[1] * invd], axis=1)
    h1 = jax.nn.relu(_dot(agg, w1l_ref[...]) + y1_ref[...])
    h1_ref[...] = h1
    p2_ref[...] = _dot(h1, w2l_ref[...])


@jax.jit
def _tc1(degp, aggx, y1, W1l, W2l):
    return pl.pallas_call(
        _tc1_body,
        grid=(_GRID,),
        in_specs=[_parts((NC, _BN, DW)), _parts((NC, _BN, 128)),
                  _rows((_BN, 256)), _full((256, 256)), _full((256, 128))],
        out_specs=[_rows((_BN, 256)), _rows((_BN, 128)), _rows((_BN, 16))],
        out_shape=[jax.ShapeDtypeStruct((N, 256), jnp.float32),
                   jax.ShapeDtypeStruct((N, 128), jnp.float32),
                   jax.ShapeDtypeStruct((N, 16), jnp.float32)],
    )(degp, aggx, y1, W1l, W2l)


def _tc2_body(ap_ref, invd_ref, y2_ref, h2_ref):
    agg = (ap_ref[0] + ap_ref[1]) * invd_ref[:, 0:1]
    h2_ref[...] = jax.nn.relu(agg + y2_ref[...])


@jax.jit
def _tc2(ap2, invd, y2):
    return pl.pallas_call(
        _tc2_body,
        grid=(_GRID,),
        in_specs=[_parts((NC, _BN, 128)), _rows((_BN, 16)),
                  _rows((_BN, 128))],
        out_specs=_rows((_BN, 128)),
        out_shape=jax.ShapeDtypeStruct((N, 128), jnp.float32),
    )(ap2, invd, y2)


def _tc3_body(ap_ref, invd_ref, y3_ref, w3l_ref, w4l_ref,
              h3_ref, p4a_ref, p4b_ref):
    agg = (ap_ref[0] + ap_ref[1]) * invd_ref[:, 0:1]
    h3 = jax.nn.relu(_dot(agg, w3l_ref[...]) + y3_ref[...])
    h3_ref[...] = h3
    p4 = _dot(h3, w4l_ref[...])
    p4a_ref[...] = p4[:, :128]
    p4b_ref[...] = p4[:, 128:]


@jax.jit
def _tc3(ah2, invd, y3, W3l, W4l):
    return pl.pallas_call(
        _tc3_body,
        grid=(_GRID,),
        in_specs=[_parts((NC, _BN, 128)), _rows((_BN, 16)),
                  _rows((_BN, 256)), _full((128, 256)), _full((256, 256))],
        out_specs=[_rows((_BN, 256)), _rows((_BN, 128)), _rows((_BN, 128))],
        out_shape=[jax.ShapeDtypeStruct((N, 256), jnp.float32),
                   jax.ShapeDtypeStruct((N, 128), jnp.float32),
                   jax.ShapeDtypeStruct((N, 128), jnp.float32)],
    )(ah2, invd, y3, W3l, W4l)


def _tc4_body(ax_ref, invd_ref, y4_ref, out_ref):
    invd = invd_ref[:, 0:1]
    agg = jnp.concatenate([ax_ref[0] * invd, ax_ref[1] * invd], axis=1)
    out_ref[...] = agg + y4_ref[...]


@jax.jit
def _tc4(agg4, invd, y4):
    return pl.pallas_call(
        _tc4_body,
        grid=(_GRID,),
        in_specs=[_parts((NC, _BN, 128)), _rows((_BN, 16)),
                  _rows((_BN, 256))],
        out_specs=_rows((_BN, 256)),
        out_shape=jax.ShapeDtypeStruct((N, 256), jnp.float32),
    )(agg4, invd, y4)


_PAD_N = EPAD - E
_PAD_SRC = jnp.asarray(np.arange(_PAD_N) % N, jnp.int32)
_PAD_DST = jnp.asarray(N + np.arange(_PAD_N) % (NPAD - N), jnp.int32)


def _parts3(flat):
    return flat.reshape(NC, NPAD, -1)


def kernel(x, edge_index, W1l, W1r, b1, W2l, W2r, b2, W3l, W3r, b3,
           W4l, W4r, b4):
    src = edge_index[0].astype(jnp.int32)
    dst = edge_index[1].astype(jnp.int32)
    src = jnp.concatenate([src, _PAD_SRC]).reshape(NCHUNK, CHUNK)
    dst = jnp.concatenate([dst, _PAD_DST]).reshape(NCHUNK, CHUNK)

    xh0 = x[:, :128]
    xh1 = x[:, 128:]

    y1 = _self_256_256(x, W1r, b1.reshape(1, -1))
    degp = _parts3(_deg_count(dst))
    aggx = _parts3(_seg_sum2(xh0, xh1, src, dst))
    h1, p2, invd = _tc1(degp, aggx, y1, W1l, W2l)

    y2 = _self_256_128(h1, W2r, b2.reshape(1, -1))
    ap2 = _parts3(_seg_sum(p2, src, dst))
    h2 = _tc2(ap2, invd, y2)

    y3 = _self_128_256(h2, W3r, b3.reshape(1, -1))
    ah2 = _parts3(_seg_sum(h2, src, dst))
    h3, p4a, p4b = _tc3(ah2, invd, y3, W3l, W4l)

    y4 = _self_256_256(h3, W4r, b4.reshape(1, -1))
    agg4 = _parts3(_seg_sum2(p4a, p4b, src, dst))
    out = _tc4(agg4, invd, y4)
    return out
